# baseline (speedup 1.0000x reference)
import functools

import jax
import jax.numpy as jnp
from jax import lax
from jax.experimental import pallas as pl
from jax.experimental.pallas import tpu as pltpu
from jax.experimental.pallas import tpu_sc as plsc

_N = 10000
_N_PAD = 10240
_R = 3
_K = 16
_D = 128
_NW = 32
_ROWS = _R * _N_PAD
_ROWS_PER_W = _ROWS // _NW
_C = 8
_CHUNKS = _ROWS_PER_W // _C
_T_PAD = 10240


def _sc_body(table_hbm, idx_hbm, out_hbm, idx_v, rows_v, acc_v, table_sp,
             *sems):
    gsems = (sems[0:2], sems[6:8])
    isems = sems[2:4]
    osems = sems[4:6]
    cid = lax.axis_index("c")
    sid = lax.axis_index("s")
    wid = sid * 2 + cid
    base = wid * _ROWS_PER_W
    ibase = wid * _CHUNKS

    tpr = _T_PAD // 16
    pltpu.sync_copy(table_hbm.at[pl.ds(sid * tpr, tpr), :],
                    table_sp.at[pl.ds(sid * tpr, tpr), :])
    pltpu.sync_copy(idx_hbm.at[pl.ds(ibase, 1), :], idx_v.at[0])
    pltpu.sync_copy(idx_hbm.at[pl.ds(ibase + 1, 1), :], idx_v.at[1])
    plsc.subcore_barrier()
    pltpu.async_copy(table_sp.at[idx_v.at[0, 0, pl.ds(0, 64)]],
                     rows_v.at[0, pl.ds(0, 64)], gsems[0][0])
    pltpu.async_copy(table_sp.at[idx_v.at[0, 0, pl.ds(64, 64)]],
                     rows_v.at[0, pl.ds(64, 64)], gsems[1][0])

    @pl.loop(0, _CHUNKS, step=2)
    def _c0(c0):
        for b in range(2):
            c = c0 + b
            nb = 1 - b

            @pl.when(c + 1 < _CHUNKS)
            def _():
                pltpu.async_copy(table_sp.at[idx_v.at[nb, 0, pl.ds(0, 64)]],
                                 rows_v.at[nb, pl.ds(0, 64)], gsems[0][nb])
                pltpu.async_copy(table_sp.at[idx_v.at[nb, 0, pl.ds(64, 64)]],
                                 rows_v.at[nb, pl.ds(64, 64)], gsems[1][nb])

            pltpu.make_async_copy(
                table_sp.at[idx_v.at[b, 0, pl.ds(0, 64)]],
                rows_v.at[b, pl.ds(0, 64)], gsems[0][b]).wait()
            pltpu.make_async_copy(
                table_sp.at[idx_v.at[b, 0, pl.ds(64, 64)]],
                rows_v.at[b, pl.ds(64, 64)], gsems[1][b]).wait()

            @pl.when(c + 2 < _CHUNKS)
            def _():
                pltpu.async_copy(idx_hbm.at[pl.ds(ibase + c + 2, 1), :],
                                 idx_v.at[b], isems[b])

            @pl.when(c >= 2)
            def _():
                pltpu.make_async_copy(
                    acc_v.at[b],
                    out_hbm.at[pl.ds(base + (c - 2) * _C, _C), :],
                    osems[b]).wait()

            @pl.loop(0, _C)
            def _acc(i):
                r0 = i * _K
                for j in range(_D // 16):
                    v = rows_v[b, r0, pl.ds(j * 16, 16)]
                    for kk in range(1, _K):
                        v = v + rows_v[b, r0 + kk, pl.ds(j * 16, 16)]
                    acc_v[b, i, pl.ds(j * 16, 16)] = v

            pltpu.async_copy(
                acc_v.at[b], out_hbm.at[pl.ds(base + c * _C, _C), :],
                osems[b])

            @pl.when(c + 2 < _CHUNKS)
            def _():
                pltpu.make_async_copy(idx_hbm.at[pl.ds(ibase + c + 2, 1), :],
                                      idx_v.at[b], isems[b]).wait()

    for b in range(2):
        pltpu.make_async_copy(
            acc_v.at[b], out_hbm.at[pl.ds(base + b * _C, _C), :],
            osems[b]).wait()


@jax.jit
def _sc_aggregate(table, idx2d):
    mesh = plsc.VectorSubcoreMesh(core_axis_name="c", subcore_axis_name="s")
    k = functools.partial(
        pl.kernel,
        out_type=jax.ShapeDtypeStruct((_ROWS, _D), jnp.float32),
        mesh=mesh,
        scratch_types=[
            pltpu.VMEM((2, 1, _C * _K), jnp.int32),
            pltpu.VMEM((2, _C * _K, _D), jnp.float32),
            pltpu.VMEM((2, _C, _D), jnp.float32),
            pltpu.VMEM_SHARED((_T_PAD, _D), jnp.float32),
        ] + [pltpu.SemaphoreType.DMA] * 8,
    )(_sc_body)
    return k(table, idx2d)


def _tc_body(agg_ref, x_ref, wr_ref, ws_ref, b_ref, o_ref):
    acc = jnp.dot(x_ref[...], ws_ref[...], preferred_element_type=jnp.float32)
    for r in range(_R):
        acc = acc + jnp.dot(agg_ref[r], wr_ref[r],
                            preferred_element_type=jnp.float32)
    o_ref[...] = jnp.maximum(acc + b_ref[...], 0.0)


def _tc_combine(agg, x_pad, wr, ws, bias2d):
    bn = 512
    return pl.pallas_call(
        _tc_body,
        grid=(_N_PAD // bn,),
        in_specs=[
            pl.BlockSpec((_R, bn, _D), lambda i: (0, i, 0)),
            pl.BlockSpec((bn, _D), lambda i: (i, 0)),
            pl.BlockSpec((_R, _D, _D), lambda i: (0, 0, 0)),
            pl.BlockSpec((_D, _D), lambda i: (0, 0)),
            pl.BlockSpec((1, _D), lambda i: (0, 0)),
        ],
        out_specs=pl.BlockSpec((bn, _D), lambda i: (i, 0)),
        out_shape=jax.ShapeDtypeStruct((_N_PAD, _D), jnp.float32),
    )(agg, x_pad, wr, ws, bias2d)


def kernel(node_features, neighbor_indices, relation_kernels, self_kernel,
           bias):
    b, n, d = node_features.shape
    x = node_features[0]
    table = jnp.concatenate(
        [jnp.zeros((1, d), x.dtype), x,
         jnp.zeros((_T_PAD - 1 - n, d), x.dtype)], axis=0)
    idx = neighbor_indices[0].astype(jnp.int32)
    idx = jnp.pad(idx, ((0, 0), (0, _N_PAD - n), (0, 0)))
    agg = _sc_aggregate(table, idx.reshape(-1, _C * _K))
    agg = agg.reshape(_R, _N_PAD, _D)
    x_pad = jnp.pad(x, ((0, _N_PAD - n), (0, 0)))
    wr = relation_kernels * (1.0 / _K)
    out = _tc_combine(agg, x_pad, wr, self_kernel, bias.reshape(1, _D))
    return out[None, :n, :]


# speedup vs baseline: 1.0093x; 1.0093x over previous
import functools

import jax
import jax.numpy as jnp
from jax import lax
from jax.experimental import pallas as pl
from jax.experimental.pallas import tpu as pltpu
from jax.experimental.pallas import tpu_sc as plsc

_N = 10000
_N_PAD = 10240
_R = 3
_K = 16
_D = 128
_NW = 32
_ROWS = _R * _N_PAD
_ROWS_PER_W = _ROWS // _NW
_C = 8
_CHUNKS = _ROWS_PER_W // _C
_T_PAD = 10240


def _sc_body(table_hbm, idx_hbm, out_hbm, idx_v, rows_v, acc_v, table_sp,
             *sems):
    gsems = sems[0:2]
    isems = sems[2:4]
    osems = sems[4:6]
    cid = lax.axis_index("c")
    sid = lax.axis_index("s")
    wid = sid * 2 + cid
    base = wid * _ROWS_PER_W
    ibase = wid * _CHUNKS

    tpr = _T_PAD // 16
    pltpu.sync_copy(table_hbm.at[pl.ds(sid * tpr, tpr), :],
                    table_sp.at[pl.ds(sid * tpr, tpr), :])
    pltpu.sync_copy(idx_hbm.at[pl.ds(ibase, 1), :], idx_v.at[0])
    pltpu.sync_copy(idx_hbm.at[pl.ds(ibase + 1, 1), :], idx_v.at[1])
    plsc.subcore_barrier()
    pltpu.async_copy(table_sp.at[idx_v.at[0, 0]], rows_v.at[0], gsems[0])

    @pl.loop(0, _CHUNKS, step=2)
    def _c0(c0):
        for b in range(2):
            c = c0 + b
            nb = 1 - b

            @pl.when(c + 1 < _CHUNKS)
            def _():
                pltpu.async_copy(table_sp.at[idx_v.at[nb, 0]], rows_v.at[nb],
                                 gsems[nb])

            pltpu.make_async_copy(
                table_sp.at[idx_v.at[b, 0]], rows_v.at[b], gsems[b]).wait()

            @pl.when(c + 2 < _CHUNKS)
            def _():
                pltpu.async_copy(idx_hbm.at[pl.ds(ibase + c + 2, 1), :],
                                 idx_v.at[b], isems[b])

            @pl.when(c >= 2)
            def _():
                pltpu.make_async_copy(
                    acc_v.at[b],
                    out_hbm.at[pl.ds(base + (c - 2) * _C, _C), :],
                    osems[b]).wait()

            @pl.loop(0, _C)
            def _acc(i):
                r0 = i * _K
                for j in range(_D // 16):
                    v = rows_v[b, r0, pl.ds(j * 16, 16)]
                    for kk in range(1, _K):
                        v = v + rows_v[b, r0 + kk, pl.ds(j * 16, 16)]
                    acc_v[b, i, pl.ds(j * 16, 16)] = v

            pltpu.async_copy(
                acc_v.at[b], out_hbm.at[pl.ds(base + c * _C, _C), :],
                osems[b])

            @pl.when(c + 2 < _CHUNKS)
            def _():
                pltpu.make_async_copy(idx_hbm.at[pl.ds(ibase + c + 2, 1), :],
                                      idx_v.at[b], isems[b]).wait()

    for b in range(2):
        pltpu.make_async_copy(
            acc_v.at[b], out_hbm.at[pl.ds(base + b * _C, _C), :],
            osems[b]).wait()


@jax.jit
def _sc_aggregate(table, idx2d):
    mesh = plsc.VectorSubcoreMesh(core_axis_name="c", subcore_axis_name="s")
    k = functools.partial(
        pl.kernel,
        out_type=jax.ShapeDtypeStruct((_ROWS, _D), jnp.float32),
        mesh=mesh,
        scratch_types=[
            pltpu.VMEM((2, 1, _C * _K), jnp.int32),
            pltpu.VMEM((2, _C * _K, _D), jnp.float32),
            pltpu.VMEM((2, _C, _D), jnp.float32),
            pltpu.VMEM_SHARED((_T_PAD, _D), jnp.float32),
        ] + [pltpu.SemaphoreType.DMA] * 6,
    )(_sc_body)
    return k(table, idx2d)


def _tc_body(agg_ref, x_ref, wr_ref, ws_ref, b_ref, o_ref):
    acc = jnp.dot(x_ref[...], ws_ref[...], preferred_element_type=jnp.float32)
    for r in range(_R):
        acc = acc + jnp.dot(agg_ref[r], wr_ref[r],
                            preferred_element_type=jnp.float32)
    o_ref[...] = jnp.maximum(acc + b_ref[...], 0.0)


def _tc_combine(agg, x, wr, ws, bias2d):
    bn = 400
    return pl.pallas_call(
        _tc_body,
        grid=(_N // bn,),
        in_specs=[
            pl.BlockSpec((_R, bn, _D), lambda i: (0, i, 0)),
            pl.BlockSpec((bn, _D), lambda i: (i, 0)),
            pl.BlockSpec((_R, _D, _D), lambda i: (0, 0, 0)),
            pl.BlockSpec((_D, _D), lambda i: (0, 0)),
            pl.BlockSpec((1, _D), lambda i: (0, 0)),
        ],
        out_specs=pl.BlockSpec((bn, _D), lambda i: (i, 0)),
        out_shape=jax.ShapeDtypeStruct((_N, _D), jnp.float32),
    )(agg, x, wr, ws, bias2d)


def kernel(node_features, neighbor_indices, relation_kernels, self_kernel,
           bias):
    b, n, d = node_features.shape
    x = node_features[0]
    table = jnp.concatenate(
        [jnp.zeros((1, d), x.dtype), x,
         jnp.zeros((_T_PAD - 1 - n, d), x.dtype)], axis=0)
    idx = neighbor_indices[0].astype(jnp.int32)
    idx = jnp.pad(idx, ((0, 0), (0, _N_PAD - n), (0, 0)))
    agg = _sc_aggregate(table, idx.reshape(-1, _C * _K))
    agg = agg.reshape(_R, _N_PAD, _D)
    wr = relation_kernels * (1.0 / _K)
    out = _tc_combine(agg, x, wr, self_kernel, bias.reshape(1, _D))
    return out[None]


# speedup vs baseline: 1.1677x; 1.1569x over previous
import functools

import jax
import jax.numpy as jnp
from jax import lax
from jax.experimental import pallas as pl
from jax.experimental.pallas import tpu as pltpu
from jax.experimental.pallas import tpu_sc as plsc

_N = 10000
_R = 3
_K = 16
_D = 128
_ROWS = _R * _N
_C = 8
_IROWS = _ROWS // _C
_CPW = 118
_T_PAD = 10240
_ZROW = 10000


def _sc_body(table_hbm, idx_hbm, out_hbm, idx_v, rows_v, acc_v, table_sp,
             *sems):
    gsems = sems[0:2]
    isems = sems[2:4]
    osems = sems[4:6]
    cid = lax.axis_index("c")
    sid = lax.axis_index("s")
    wid = sid * 2 + cid
    base = wid * _CPW * _C
    ibase = wid * _CPW
    nchunks = jnp.minimum(_CPW, _IROWS - wid * _CPW)

    tpr = _T_PAD // 16

    @pl.when(sid < 15)
    def _():
        pltpu.sync_copy(table_hbm.at[pl.ds(sid * tpr, tpr), :],
                        table_sp.at[pl.ds(sid * tpr, tpr), :])

    @pl.when(sid == 15)
    def _():
        pltpu.sync_copy(table_hbm.at[pl.ds(15 * tpr, _N - 15 * tpr), :],
                        table_sp.at[pl.ds(15 * tpr, _N - 15 * tpr), :])
        zero = jnp.zeros((16,), jnp.float32)
        for i in range(_C):
            for j in range(_D // 16):
                acc_v[0, i, pl.ds(j * 16, 16)] = zero
        pltpu.sync_copy(acc_v.at[0], table_sp.at[pl.ds(_ZROW, _C), :])

    pltpu.sync_copy(idx_hbm.at[pl.ds(ibase * 128, 128)], idx_v.at[0])
    pltpu.sync_copy(idx_hbm.at[pl.ds((ibase + 1) * 128, 128)], idx_v.at[1])
    plsc.subcore_barrier()
    pltpu.async_copy(table_sp.at[idx_v.at[0]], rows_v.at[0], gsems[0])

    @pl.loop(0, nchunks, step=2)
    def _c0(c0):
        for b in range(2):
            c = c0 + b
            nb = 1 - b

            @pl.when(c + 1 < nchunks)
            def _():
                pltpu.async_copy(table_sp.at[idx_v.at[nb]], rows_v.at[nb],
                                 gsems[nb])

            pltpu.make_async_copy(
                table_sp.at[idx_v.at[b]], rows_v.at[b], gsems[b]).wait()

            @pl.when(c + 2 < nchunks)
            def _():
                pltpu.async_copy(idx_hbm.at[pl.ds((ibase + c + 2) * 128, 128)],
                                 idx_v.at[b], isems[b])

            @pl.when(c >= 2)
            def _():
                pltpu.make_async_copy(
                    acc_v.at[b],
                    out_hbm.at[pl.ds(base + (c - 2) * _C, _C), :],
                    osems[b]).wait()

            @pl.loop(0, _C)
            def _acc(i):
                r0 = i * _K
                for j in range(_D // 16):
                    v = rows_v[b, r0, pl.ds(j * 16, 16)]
                    for kk in range(1, _K):
                        v = v + rows_v[b, r0 + kk, pl.ds(j * 16, 16)]
                    acc_v[b, i, pl.ds(j * 16, 16)] = v

            pltpu.async_copy(
                acc_v.at[b], out_hbm.at[pl.ds(base + c * _C, _C), :],
                osems[b])

            @pl.when(c + 2 < nchunks)
            def _():
                pltpu.make_async_copy(
                    idx_hbm.at[pl.ds((ibase + c + 2) * 128, 128)],
                    idx_v.at[b], isems[b]).wait()

    for b in range(2):
        pltpu.make_async_copy(
            acc_v.at[b], out_hbm.at[pl.ds(base + b * _C, _C), :],
            osems[b]).wait()


@jax.jit
def _sc_aggregate(table, idx2d):
    mesh = plsc.VectorSubcoreMesh(core_axis_name="c", subcore_axis_name="s")
    k = functools.partial(
        pl.kernel,
        out_type=jax.ShapeDtypeStruct((_ROWS, _D), jnp.float32),
        mesh=mesh,
        scratch_types=[
            pltpu.VMEM((2, _C * _K), jnp.int32),
            pltpu.VMEM((2, _C * _K, _D), jnp.float32),
            pltpu.VMEM((2, _C, _D), jnp.float32),
            pltpu.VMEM_SHARED((_T_PAD, _D), jnp.float32),
        ] + [pltpu.SemaphoreType.DMA] * 6,
    )(_sc_body)
    return k(table, idx2d)


def _tc_body(agg_ref, x_ref, wr_ref, ws_ref, b_ref, o_ref):
    acc = jnp.dot(x_ref[...], ws_ref[...], preferred_element_type=jnp.float32)
    for r in range(_R):
        acc = acc + jnp.dot(agg_ref[r], wr_ref[r],
                            preferred_element_type=jnp.float32)
    o_ref[...] = jnp.maximum(acc + b_ref[...], 0.0)


def _tc_combine(agg, x, wr, ws, bias2d):
    bn = 1000
    return pl.pallas_call(
        _tc_body,
        grid=(_N // bn,),
        in_specs=[
            pl.BlockSpec((_R, bn, _D), lambda i: (0, i, 0)),
            pl.BlockSpec((bn, _D), lambda i: (i, 0)),
            pl.BlockSpec((_R, _D, _D), lambda i: (0, 0, 0)),
            pl.BlockSpec((_D, _D), lambda i: (0, 0)),
            pl.BlockSpec((1, _D), lambda i: (0, 0)),
        ],
        out_specs=pl.BlockSpec((bn, _D), lambda i: (i, 0)),
        out_shape=jax.ShapeDtypeStruct((_N, _D), jnp.float32),
    )(agg, x, wr, ws, bias2d)


def kernel(node_features, neighbor_indices, relation_kernels, self_kernel,
           bias):
    b, n, d = node_features.shape
    x = node_features[0]
    idx = neighbor_indices[0].astype(jnp.int32)
    idxm = jnp.where(idx == 0, _ZROW, idx - 1).reshape(-1)
    agg = _sc_aggregate(x, idxm)
    agg = agg.reshape(_R, _N, _D)
    wr = relation_kernels * (1.0 / _K)
    out = _tc_combine(agg, x, wr, self_kernel, bias.reshape(1, _D))
    return out[None]


# speedup vs baseline: 1.2172x; 1.0425x over previous
import functools

import jax
import jax.numpy as jnp
from jax import lax
from jax.experimental import pallas as pl
from jax.experimental.pallas import tpu as pltpu
from jax.experimental.pallas import tpu_sc as plsc

_N = 10000
_R = 3
_K = 16
_D = 128
_ROWS = _R * _N
_C = 8
_IROWS = _ROWS // _C
_CPW = 118
_T_PAD = 10240
_ZROW = 10000


def _sc_body(table_hbm, idx_hbm, out_hbm, idx_r, idx_v, rows_v, acc_v,
             table_sp, *sems):
    gsems = sems[0:2]
    isems = sems[2:4]
    osems = sems[4:6]
    cid = lax.axis_index("c")
    sid = lax.axis_index("s")
    wid = sid * 2 + cid
    base = wid * _CPW * _C
    ibase = wid * _CPW
    nchunks = jnp.minimum(_CPW, _IROWS - wid * _CPW)

    tpr = _T_PAD // 16

    @pl.when(sid < 15)
    def _():
        pltpu.sync_copy(table_hbm.at[pl.ds(sid * tpr, tpr), :],
                        table_sp.at[pl.ds(sid * tpr, tpr), :])

    @pl.when(sid == 15)
    def _():
        pltpu.sync_copy(table_hbm.at[pl.ds(15 * tpr, _N - 15 * tpr), :],
                        table_sp.at[pl.ds(15 * tpr, _N - 15 * tpr), :])
        zero = jnp.zeros((16,), jnp.float32)
        for i in range(_C):
            for j in range(_D // 16):
                acc_v[0, i, pl.ds(j * 16, 16)] = zero
        pltpu.sync_copy(acc_v.at[0], table_sp.at[pl.ds(_ZROW, _C), :])

    def idx_src(c):
        r = c // (_N // _C)
        node0 = (c % (_N // _C)) * _C
        return idx_hbm.at[r, pl.ds(node0, _C), :]

    def remap(bb):
        for q in range(_C):
            v = idx_r[bb, q, :]
            idx_v[bb, 0, pl.ds(q * _K, _K)] = jnp.where(v == 0, _ZROW, v - 1)

    pltpu.sync_copy(idx_src(ibase), idx_r.at[0])
    pltpu.sync_copy(idx_src(ibase + 1), idx_r.at[1])
    remap(0)
    remap(1)
    plsc.subcore_barrier()
    pltpu.async_copy(table_sp.at[idx_v.at[0, 0]], rows_v.at[0], gsems[0])

    @pl.loop(0, nchunks, step=2)
    def _c0(c0):
        for b in range(2):
            c = c0 + b
            nb = 1 - b

            @pl.when(c + 1 < nchunks)
            def _():
                pltpu.async_copy(table_sp.at[idx_v.at[nb, 0]], rows_v.at[nb],
                                 gsems[nb])

            pltpu.make_async_copy(
                table_sp.at[idx_v.at[b, 0]], rows_v.at[b], gsems[b]).wait()

            @pl.when(c + 2 < nchunks)
            def _():
                pltpu.async_copy(idx_src(ibase + c + 2), idx_r.at[b],
                                 isems[b])

            @pl.when(c >= 2)
            def _():
                pltpu.make_async_copy(
                    acc_v.at[b],
                    out_hbm.at[pl.ds(base + (c - 2) * _C, _C), :],
                    osems[b]).wait()

            @pl.loop(0, _C)
            def _acc(i):
                r0 = i * _K
                for j in range(_D // 16):
                    v = rows_v[b, r0, pl.ds(j * 16, 16)]
                    for kk in range(1, _K):
                        v = v + rows_v[b, r0 + kk, pl.ds(j * 16, 16)]
                    acc_v[b, i, pl.ds(j * 16, 16)] = v

            pltpu.async_copy(
                acc_v.at[b], out_hbm.at[pl.ds(base + c * _C, _C), :],
                osems[b])

            @pl.when(c + 2 < nchunks)
            def _():
                pltpu.make_async_copy(
                    idx_src(ibase + c + 2), idx_r.at[b], isems[b]).wait()
                remap(b)

    for b in range(2):
        pltpu.make_async_copy(
            acc_v.at[b], out_hbm.at[pl.ds(base + b * _C, _C), :],
            osems[b]).wait()


@jax.jit
def _sc_aggregate(table, idx2d):
    mesh = plsc.VectorSubcoreMesh(core_axis_name="c", subcore_axis_name="s")
    k = functools.partial(
        pl.kernel,
        out_type=jax.ShapeDtypeStruct((_ROWS, _D), jnp.float32),
        mesh=mesh,
        scratch_types=[
            pltpu.VMEM((2, _C, _K), jnp.int32),
            pltpu.VMEM((2, 1, _C * _K), jnp.int32),
            pltpu.VMEM((2, _C * _K, _D), jnp.float32),
            pltpu.VMEM((2, _C, _D), jnp.float32),
            pltpu.VMEM_SHARED((_T_PAD, _D), jnp.float32),
        ] + [pltpu.SemaphoreType.DMA] * 6,
    )(_sc_body)
    return k(table, idx2d)


def _tc_body(agg_ref, x_ref, wr_ref, ws_ref, b_ref, o_ref):
    acc = jnp.dot(x_ref[...], ws_ref[...], preferred_element_type=jnp.float32)
    for r in range(_R):
        acc = acc + jnp.dot(agg_ref[r], wr_ref[r],
                            preferred_element_type=jnp.float32)
    o_ref[...] = jnp.maximum(acc + b_ref[...], 0.0)


def _tc_combine(agg, x, wr, ws, bias2d):
    bn = 1000
    return pl.pallas_call(
        _tc_body,
        grid=(_N // bn,),
        in_specs=[
            pl.BlockSpec((_R, bn, _D), lambda i: (0, i, 0)),
            pl.BlockSpec((bn, _D), lambda i: (i, 0)),
            pl.BlockSpec((_R, _D, _D), lambda i: (0, 0, 0)),
            pl.BlockSpec((_D, _D), lambda i: (0, 0)),
            pl.BlockSpec((1, _D), lambda i: (0, 0)),
        ],
        out_specs=pl.BlockSpec((bn, _D), lambda i: (i, 0)),
        out_shape=jax.ShapeDtypeStruct((_N, _D), jnp.float32),
    )(agg, x, wr, ws, bias2d)


def kernel(node_features, neighbor_indices, relation_kernels, self_kernel,
           bias):
    b, n, d = node_features.shape
    x = node_features[0]
    idx = neighbor_indices[0].astype(jnp.int32)
    agg = _sc_aggregate(x, idx)
    agg = agg.reshape(_R, _N, _D)
    wr = relation_kernels * (1.0 / _K)
    out = _tc_combine(agg, x, wr, self_kernel, bias.reshape(1, _D))
    return out[None]


# speedup vs baseline: 1.2382x; 1.0172x over previous
import functools

import jax
import jax.numpy as jnp
from jax import lax
from jax.experimental import pallas as pl
from jax.experimental.pallas import tpu as pltpu
from jax.experimental.pallas import tpu_sc as plsc

_N = 10000
_R = 3
_K = 16
_D = 128
_ROWS = _R * _N
_C = 8
_IROWS = _ROWS // _C
_CPW = 118
_T_PAD = 10240
_ZROW = 10000


def _sc_body(table_hbm, idx_hbm, out_hbm, idx_r, idx_v, rows_v, acc_v,
             table_sp, *sems):
    gsems = sems[0:2]
    isems = sems[2:4]
    osems = sems[4:6]
    cid = lax.axis_index("c")
    sid = lax.axis_index("s")
    wid = sid * 2 + cid
    base = wid * _CPW * _C
    ibase = wid * _CPW
    nchunks = jnp.minimum(_CPW, _IROWS - wid * _CPW)

    tpr = _T_PAD // 16

    @pl.when(sid < 15)
    def _():
        pltpu.sync_copy(table_hbm.at[pl.ds(sid * tpr, tpr), :],
                        table_sp.at[pl.ds(sid * tpr, tpr), :])

    @pl.when(sid == 15)
    def _():
        pltpu.sync_copy(table_hbm.at[pl.ds(15 * tpr, _N - 15 * tpr), :],
                        table_sp.at[pl.ds(15 * tpr, _N - 15 * tpr), :])
        zero = jnp.zeros((16,), jnp.float32)
        for i in range(_C):
            for j in range(_D // 16):
                acc_v[0, i, pl.ds(j * 16, 16)] = zero
        pltpu.sync_copy(acc_v.at[0], table_sp.at[pl.ds(_ZROW, _C), :])

    def idx_src(c):
        r = c // (_N // _C)
        node0 = (c % (_N // _C)) * _C
        return idx_hbm.at[r, pl.ds(node0, _C), :]

    def remap(bb):
        for q in range(_C):
            v = idx_r[bb, q, :]
            idx_v[bb, 0, pl.ds(q * _K, _K)] = jnp.where(v == 0, _ZROW, v - 1)

    pltpu.sync_copy(idx_src(ibase), idx_r.at[0])
    pltpu.sync_copy(idx_src(ibase + 1), idx_r.at[1])
    remap(0)
    remap(1)
    plsc.subcore_barrier()
    pltpu.async_copy(table_sp.at[idx_v.at[0, 0]], rows_v.at[0], gsems[0])

    @pl.loop(0, nchunks, step=2)
    def _c0(c0):
        for b in range(2):
            c = c0 + b
            nb = 1 - b

            @pl.when(c + 1 < nchunks)
            def _():
                pltpu.async_copy(table_sp.at[idx_v.at[nb, 0]], rows_v.at[nb],
                                 gsems[nb])

            pltpu.make_async_copy(
                table_sp.at[idx_v.at[b, 0]], rows_v.at[b], gsems[b]).wait()

            @pl.when(c + 2 < nchunks)
            def _():
                pltpu.async_copy(idx_src(ibase + c + 2), idx_r.at[b],
                                 isems[b])

            @pl.when(c >= 2)
            def _():
                pltpu.make_async_copy(
                    acc_v.at[b],
                    out_hbm.at[pl.ds(base + (c - 2) * _C, _C), :],
                    osems[b]).wait()

            @pl.loop(0, _C)
            def _acc(i):
                r0 = i * _K
                for j in range(_D // 16):
                    v = rows_v[b, r0, pl.ds(j * 16, 16)]
                    for kk in range(1, _K):
                        v = v + rows_v[b, r0 + kk, pl.ds(j * 16, 16)]
                    acc_v[b, i, pl.ds(j * 16, 16)] = v

            pltpu.async_copy(
                acc_v.at[b], out_hbm.at[pl.ds(base + c * _C, _C), :],
                osems[b])

            @pl.when(c + 2 < nchunks)
            def _():
                pltpu.make_async_copy(
                    idx_src(ibase + c + 2), idx_r.at[b], isems[b]).wait()
                remap(b)

    for b in range(2):
        pltpu.make_async_copy(
            acc_v.at[b], out_hbm.at[pl.ds(base + b * _C, _C), :],
            osems[b]).wait()


@jax.jit
def _sc_aggregate(table, idx2d):
    mesh = plsc.VectorSubcoreMesh(core_axis_name="c", subcore_axis_name="s")
    k = functools.partial(
        pl.kernel,
        out_type=jax.ShapeDtypeStruct((_ROWS, _D), jnp.float32),
        mesh=mesh,
        scratch_types=[
            pltpu.VMEM((2, _C, _K), jnp.int32),
            pltpu.VMEM((2, 1, _C * _K), jnp.int32),
            pltpu.VMEM((2, _C * _K, _D), jnp.float32),
            pltpu.VMEM((2, _C, _D), jnp.float32),
            pltpu.VMEM_SHARED((_T_PAD, _D), jnp.float32),
        ] + [pltpu.SemaphoreType.DMA] * 6,
    )(_sc_body)
    return k(table, idx2d)


def _tc_body(agg_ref, x_ref, wr_ref, ws_ref, b_ref, o_ref):
    acc = jnp.dot(x_ref[...], ws_ref[...], preferred_element_type=jnp.float32)
    for r in range(_R):
        acc = acc + jnp.dot(agg_ref[r], wr_ref[r],
                            preferred_element_type=jnp.float32)
    o_ref[...] = jnp.maximum(acc + b_ref[...], 0.0)


def _tc_combine(agg, x, wr, ws, bias2d):
    bn = 2000
    return pl.pallas_call(
        _tc_body,
        grid=(_N // bn,),
        in_specs=[
            pl.BlockSpec((_R, bn, _D), lambda i: (0, i, 0)),
            pl.BlockSpec((bn, _D), lambda i: (i, 0)),
            pl.BlockSpec((_R, _D, _D), lambda i: (0, 0, 0)),
            pl.BlockSpec((_D, _D), lambda i: (0, 0)),
            pl.BlockSpec((1, _D), lambda i: (0, 0)),
        ],
        out_specs=pl.BlockSpec((bn, _D), lambda i: (i, 0)),
        out_shape=jax.ShapeDtypeStruct((_N, _D), jnp.float32),
    )(agg, x, wr, ws, bias2d)


def kernel(node_features, neighbor_indices, relation_kernels, self_kernel,
           bias):
    b, n, d = node_features.shape
    x = node_features[0]
    idx = neighbor_indices[0].astype(jnp.int32)
    agg = _sc_aggregate(x, idx)
    agg = agg.reshape(_R, _N, _D)
    wr = relation_kernels * (1.0 / _K)
    out = _tc_combine(agg, x, wr, self_kernel, bias.reshape(1, _D))
    return out[None]
